# baseline (speedup 1.0000x reference)
import functools

import jax
import jax.numpy as jnp
from jax import lax
from jax.experimental import pallas as pl
from jax.experimental.pallas import tpu as pltpu
from jax.experimental.pallas import tpu_sc as plsc

N, E, H, D, C = 10000, 320000, 8, 16, 10
F = H * D
FP = F
LANES = 16
NC, NS = 2, 16
NW = NC * NS
K = 128
NCHUNK = E // K
LASTC = NCHUNK - 1
GROUPS = K // LANES
NITER = -(-NCHUNK // NW)
NITER += NITER % 2


def _sqrt16(x):
    i = lax.bitcast_convert_type(x, jnp.int32)
    i = jnp.int32(0x5F3759DF) - lax.shift_right_arithmetic(i, 1)
    y = lax.bitcast_convert_type(i, jnp.float32)
    for _ in range(3):
        y = y * (1.5 - 0.5 * x * y * y)
    return x * y


def _body(h_ref, sd_ref, w_ref, r_ref, score_out, std_out,
          sd0, sd1, srows0, drows0, srows1, drows1,
          score0, std0, score1, std1,
          wt_scr, ub_scr, semi0, semi1, semg0, semg1, semo0, semo1):
    wid = lax.axis_index("s") * NC + lax.axis_index("c")

    pltpu.sync_copy(w_ref, wt_scr)
    pltpu.sync_copy(r_ref, ub_scr)
    wv = wt_scr[...]
    ev = jnp.exp(wv - jnp.max(wv))
    weight = ev / jnp.sum(ev)
    rv = ub_scr[...]
    ra = jnp.abs(rv)
    cs = plsc.cumsum(ra)
    ub_v = cs + 0.1
    lb_v = cs - ra
    inv_v = 1.0 / (rv * rv + 1.0)
    wgt = [weight[hh] for hh in range(H)]
    ubs = [ub_v[c] for c in range(C)]
    lbs = [lb_v[c] for c in range(C)]
    invs = [inv_v[c] for c in range(C)]

    lane = lax.iota(jnp.int32, LANES)
    rots = [jnp.bitwise_and(lane + t, LANES - 1) for t in range(D)]
    csplats10 = [jnp.full((LANES,), c, jnp.int32) for c in range(C)]

    sds = (sd0, sd1)
    rows = ((srows0, drows0), (srows1, drows1))
    outs = ((score0, std0), (score1, std1))
    semi = (semi0, semi1)
    semg = (semg0, semg1)
    semo = (semo0, semo1)

    def cid(i):
        return jnp.minimum(wid + i * NW, LASTC)

    def issue_idx(i, p):
        pltpu.async_copy(sd_ref.at[cid(i)], sds[p], semi[p])

    def wait_idx(i, p):
        pltpu.make_async_copy(sd_ref.at[cid(i)], sds[p], semi[p]).wait()

    def issue_gather(p):
        pltpu.async_copy(h_ref.at[sds[p].at[0]], rows[p][0], semg[p])
        pltpu.async_copy(h_ref.at[sds[p].at[1]], rows[p][1], semg[p])

    def wait_gather(p):
        pltpu.make_async_copy(h_ref.at[sds[p].at[0]], rows[p][0], semg[p]).wait()
        pltpu.make_async_copy(h_ref.at[sds[p].at[1]], rows[p][1], semg[p]).wait()

    def compute(i, p):
        srows, drows = rows[p]
        score_scr, std_scr = outs[p]
        base = cid(i) * K

        def group_body(g, _):
            e_vec = lane + g * LANES
            n2 = []
            for hh in range(H):
                acc = None
                for tt in range(D):
                    col = rots[tt] + (hh * D)
                    sv = plsc.load_gather(srows, [e_vec, col])
                    dv = plsc.load_gather(drows, [e_vec, col])
                    df = dv - sv
                    acc = df * df if acc is None else acc + df * df
                n2.append(_sqrt16(acc))
            dist = n2[0] * wgt[0]
            for hh in range(1, H):
                dist = dist + n2[hh] * wgt[hh]
            sacc = None
            for hh in range(H):
                rr = n2[hh] - dist
                sacc = rr * rr if sacc is None else sacc + rr * rr
            std_scr[pl.ds(g * LANES, LANES)] = _sqrt16(sacc * (1.0 / H))
            for c in range(C):
                val = (ubs[c] - dist) * (dist - lbs[c]) * invs[c]
                plsc.store_scatter(score_scr, [e_vec, csplats10[c]], val)
            return 0

        lax.fori_loop(0, GROUPS, group_body, 0, unroll=2)
        pltpu.async_copy(score_scr, score_out.at[pl.ds(base, K)], semo[p])
        pltpu.async_copy(std_scr, std_out.at[pl.ds(base, K)], semo[p])

    def wait_out(i, p):
        base = cid(i) * K
        pltpu.make_async_copy(
            outs[p][0], score_out.at[pl.ds(base, K)], semo[p]).wait()
        pltpu.make_async_copy(
            outs[p][1], std_out.at[pl.ds(base, K)], semo[p]).wait()

    issue_idx(0, 0)
    issue_idx(1, 1)
    wait_idx(0, 0)
    issue_gather(0)

    def pair_body(j, _):
        for p in (0, 1):
            i = 2 * j + p
            wait_gather(p)
            wait_idx(i + 1, 1 - p)
            issue_gather(1 - p)
            issue_idx(i + 2, p)

            @pl.when(j >= 1)
            def _():
                wait_out(i - 2, p)

            compute(i, p)
        return 0

    lax.fori_loop(0, NITER // 2, pair_body, 0, unroll=1)
    wait_gather(NITER % 2)
    wait_idx(NITER + 1, (NITER + 1) % 2)
    wait_out(NITER - 2, 0)
    wait_out(NITER - 1, 1)


@jax.jit
def kernel(h, edge_index, w, r_dist):
    h2 = h.reshape(N, F)
    sd = edge_index.reshape(2, NCHUNK, K).transpose(1, 0, 2)
    w16 = jnp.full((LANES,), -1e30, jnp.float32).at[:H].set(w)
    r16 = jnp.zeros((LANES,), jnp.float32).at[:C].set(r_dist[0])

    mesh = plsc.VectorSubcoreMesh(
        core_axis_name="c", subcore_axis_name="s",
        num_cores=NC, num_subcores=NS)
    f = pl.kernel(
        _body,
        out_type=[
            jax.ShapeDtypeStruct((E, C), jnp.float32),
            jax.ShapeDtypeStruct((E,), jnp.float32),
        ],
        mesh=mesh,
        compiler_params=pltpu.CompilerParams(needs_layout_passes=False),
        scratch_types=[
            pltpu.VMEM((2, K), jnp.int32),
            pltpu.VMEM((2, K), jnp.int32),
            pltpu.VMEM((K, FP), jnp.float32),
            pltpu.VMEM((K, FP), jnp.float32),
            pltpu.VMEM((K, FP), jnp.float32),
            pltpu.VMEM((K, FP), jnp.float32),
            pltpu.VMEM((K, C), jnp.float32),
            pltpu.VMEM((K,), jnp.float32),
            pltpu.VMEM((K, C), jnp.float32),
            pltpu.VMEM((K,), jnp.float32),
            pltpu.VMEM((LANES,), jnp.float32),
            pltpu.VMEM((LANES,), jnp.float32),
            pltpu.SemaphoreType.DMA,
            pltpu.SemaphoreType.DMA,
            pltpu.SemaphoreType.DMA,
            pltpu.SemaphoreType.DMA,
            pltpu.SemaphoreType.DMA,
            pltpu.SemaphoreType.DMA,
        ],
    )
    score, std = f(h2, sd, w16, r16)
    return score, std.reshape(E, 1)


# speedup vs baseline: 1.0906x; 1.0906x over previous
import functools

import jax
import jax.numpy as jnp
from jax import lax
from jax.experimental import pallas as pl
from jax.experimental.pallas import tpu as pltpu
from jax.experimental.pallas import tpu_sc as plsc

N, E, H, D, C = 10000, 320000, 8, 16, 10
F = H * D
FP = F
LANES = 16
NC, NS = 2, 16
NW = NC * NS
K = 128
NCHUNK = E // K
LASTC = NCHUNK - 1
GROUPS = K // LANES
NITER = -(-NCHUNK // NW)
NITER += NITER % 2


def _sqrt16(x):
    i = lax.bitcast_convert_type(x, jnp.int32)
    i = jnp.int32(0x5F3759DF) - lax.shift_right_arithmetic(i, 1)
    y = lax.bitcast_convert_type(i, jnp.float32)
    for _ in range(3):
        y = y * (1.5 - 0.5 * x * y * y)
    return x * y


def _body(h_ref, sd_ref, w_ref, r_ref, score_out, std_out,
          sd0, sd1, srows0, drows0, srows1, drows1,
          score0, std0, score1, std1,
          wt_scr, ub_scr, semi0, semi1, semg0, semg1, semo0, semo1):
    wid = lax.axis_index("s") * NC + lax.axis_index("c")

    pltpu.sync_copy(w_ref, wt_scr)
    pltpu.sync_copy(r_ref, ub_scr)
    wv = wt_scr[...]
    ev = jnp.exp(wv - jnp.max(wv))
    weight = ev / jnp.sum(ev)
    rv = ub_scr[...]
    ra = jnp.abs(rv)
    cs = plsc.cumsum(ra)
    ub_v = cs + 0.1
    lb_v = cs - ra
    inv_v = 1.0 / (rv * rv + 1.0)
    wgt = [weight[hh] for hh in range(H)]
    ubs = [ub_v[c] for c in range(C)]
    lbs = [lb_v[c] for c in range(C)]
    invs = [inv_v[c] for c in range(C)]

    lane = lax.iota(jnp.int32, LANES)
    lane2 = lane * 2
    csplats10 = [jnp.full((LANES,), c, jnp.int32) for c in range(C)]

    sds = (sd0, sd1)
    rows = ((srows0, drows0), (srows1, drows1))
    outs = ((score0, std0), (score1, std1))
    semi = (semi0, semi1)
    semg = (semg0, semg1)
    semo = (semo0, semo1)

    def cid(i):
        return jnp.minimum(wid + i * NW, LASTC)

    def issue_idx(i, p):
        pltpu.async_copy(sd_ref.at[cid(i)], sds[p], semi[p])

    def wait_idx(i, p):
        pltpu.make_async_copy(sd_ref.at[cid(i)], sds[p], semi[p]).wait()

    def issue_gather(p):
        pltpu.async_copy(h_ref.at[sds[p].at[0]], rows[p][0], semg[p])
        pltpu.async_copy(h_ref.at[sds[p].at[1]], rows[p][1], semg[p])

    def wait_gather(p):
        pltpu.make_async_copy(h_ref.at[sds[p].at[0]], rows[p][0], semg[p]).wait()
        pltpu.make_async_copy(h_ref.at[sds[p].at[1]], rows[p][1], semg[p]).wait()

    def compute(i, p):
        srows, drows = rows[p]
        score_scr, std_scr = outs[p]
        base = cid(i) * K

        def group_body(g, _):
            e_vec = lane + g * LANES
            n2 = []
            for h2 in range(H // 2):
                acc_a = acc_b = None
                for tt in range(D):
                    w1 = jnp.bitwise_and(lane2 + tt, 2 * D - 1)
                    w2 = jnp.bitwise_xor(w1, D)
                    c1 = w1 + (h2 * 2 * D)
                    c2 = w2 + (h2 * 2 * D)
                    sv1 = plsc.load_gather(srows, [e_vec, c1])
                    dv1 = plsc.load_gather(drows, [e_vec, c1])
                    sv2 = plsc.load_gather(srows, [e_vec, c2])
                    dv2 = plsc.load_gather(drows, [e_vec, c2])
                    df1 = dv1 - sv1
                    sq1 = df1 * df1
                    df2 = dv2 - sv2
                    sq2 = df2 * df2
                    in_a = w1 < D
                    a = jnp.where(in_a, sq1, sq2)
                    b = jnp.where(in_a, sq2, sq1)
                    acc_a = a if acc_a is None else acc_a + a
                    acc_b = b if acc_b is None else acc_b + b
                n2.append(_sqrt16(acc_a))
                n2.append(_sqrt16(acc_b))
            dist = n2[0] * wgt[0]
            for hh in range(1, H):
                dist = dist + n2[hh] * wgt[hh]
            sacc = None
            for hh in range(H):
                rr = n2[hh] - dist
                sacc = rr * rr if sacc is None else sacc + rr * rr
            std_scr[pl.ds(g * LANES, LANES)] = _sqrt16(sacc * (1.0 / H))
            for c in range(C):
                val = (ubs[c] - dist) * (dist - lbs[c]) * invs[c]
                plsc.store_scatter(score_scr, [e_vec, csplats10[c]], val)
            return 0

        lax.fori_loop(0, GROUPS, group_body, 0, unroll=1)
        pltpu.async_copy(score_scr, score_out.at[pl.ds(base, K)], semo[p])
        pltpu.async_copy(std_scr, std_out.at[pl.ds(base, K)], semo[p])

    def wait_out(i, p):
        base = cid(i) * K
        pltpu.make_async_copy(
            outs[p][0], score_out.at[pl.ds(base, K)], semo[p]).wait()
        pltpu.make_async_copy(
            outs[p][1], std_out.at[pl.ds(base, K)], semo[p]).wait()

    issue_idx(0, 0)
    issue_idx(1, 1)
    wait_idx(0, 0)
    issue_gather(0)

    def pair_body(j, _):
        for p in (0, 1):
            i = 2 * j + p
            wait_gather(p)
            wait_idx(i + 1, 1 - p)
            issue_gather(1 - p)
            issue_idx(i + 2, p)

            @pl.when(j >= 1)
            def _():
                wait_out(i - 2, p)

            compute(i, p)
        return 0

    lax.fori_loop(0, NITER // 2, pair_body, 0, unroll=1)
    wait_gather(NITER % 2)
    wait_idx(NITER + 1, (NITER + 1) % 2)
    wait_out(NITER - 2, 0)
    wait_out(NITER - 1, 1)


@jax.jit
def kernel(h, edge_index, w, r_dist):
    h2 = h.reshape(N, F)
    sd = edge_index.reshape(2, NCHUNK, K).transpose(1, 0, 2)
    w16 = jnp.full((LANES,), -1e30, jnp.float32).at[:H].set(w)
    r16 = jnp.zeros((LANES,), jnp.float32).at[:C].set(r_dist[0])

    mesh = plsc.VectorSubcoreMesh(
        core_axis_name="c", subcore_axis_name="s",
        num_cores=NC, num_subcores=NS)
    f = pl.kernel(
        _body,
        out_type=[
            jax.ShapeDtypeStruct((E, C), jnp.float32),
            jax.ShapeDtypeStruct((E,), jnp.float32),
        ],
        mesh=mesh,
        compiler_params=pltpu.CompilerParams(needs_layout_passes=False),
        scratch_types=[
            pltpu.VMEM((2, K), jnp.int32),
            pltpu.VMEM((2, K), jnp.int32),
            pltpu.VMEM((K, FP), jnp.float32),
            pltpu.VMEM((K, FP), jnp.float32),
            pltpu.VMEM((K, FP), jnp.float32),
            pltpu.VMEM((K, FP), jnp.float32),
            pltpu.VMEM((K, C), jnp.float32),
            pltpu.VMEM((K,), jnp.float32),
            pltpu.VMEM((K, C), jnp.float32),
            pltpu.VMEM((K,), jnp.float32),
            pltpu.VMEM((LANES,), jnp.float32),
            pltpu.VMEM((LANES,), jnp.float32),
            pltpu.SemaphoreType.DMA,
            pltpu.SemaphoreType.DMA,
            pltpu.SemaphoreType.DMA,
            pltpu.SemaphoreType.DMA,
            pltpu.SemaphoreType.DMA,
            pltpu.SemaphoreType.DMA,
        ],
    )
    score, std = f(h2, sd, w16, r16)
    return score, std.reshape(E, 1)
